# baseline (speedup 1.0000x reference)
import functools

import jax
import jax.numpy as jnp
from jax import lax
from jax.experimental import pallas as pl
from jax.experimental.pallas import tpu as pltpu
from jax.experimental.pallas import tpu_sc as plsc

N = 10000
E = 320000
D = 128

NC = 2
NS = 16
NW = NC * NS
EPT = E // NW
CH = 80
NCHUNK = EPT // CH

ACH = 80
ANCHUNK = 125
N_PAD = 10240
RPT = N_PAD // NS

_MESH = plsc.VectorSubcoreMesh(core_axis_name="c", subcore_axis_name="s")


@functools.partial(
    pl.kernel,
    out_type=jax.ShapeDtypeStruct((NC, N_PAD), jnp.float32),
    mesh=_MESH,
    scratch_types=[
        pltpu.VMEM((NCHUNK, CH), jnp.int32),
        pltpu.VMEM((CH,), jnp.float32),
        pltpu.VMEM_SHARED((N_PAD,), jnp.float32),
    ],
)
def _deg_kernel(dst_hbm, z1_hbm, ones_hbm, out_hbm, idx_v, ones_v, deg_sh):
    c = lax.axis_index("c")
    s = lax.axis_index("s")
    wid = c * NS + s
    pltpu.sync_copy(z1_hbm.at[pl.ds(s * RPT, RPT)],
                    deg_sh.at[pl.ds(s * RPT, RPT)])
    pltpu.sync_copy(dst_hbm.at[wid], idx_v)
    pltpu.sync_copy(ones_hbm, ones_v)
    plsc.subcore_barrier()

    def body(j, carry):
        pltpu.sync_copy(ones_v, deg_sh.at[idx_v.at[j]], add=True)
        return carry

    lax.fori_loop(0, NCHUNK, body, 0)
    plsc.subcore_barrier()
    pltpu.sync_copy(deg_sh.at[pl.ds(s * RPT, RPT)],
                    out_hbm.at[c, pl.ds(s * RPT, RPT)])


@functools.partial(
    pl.kernel,
    out_type=jax.ShapeDtypeStruct((NC, N_PAD, D), jnp.float32),
    mesh=_MESH,
    scratch_types=[
        pltpu.VMEM((ANCHUNK, ACH), jnp.int32),
        pltpu.VMEM((ACH,), jnp.int32),
        pltpu.VMEM((ACH,), jnp.int32),
        pltpu.VMEM((ACH,), jnp.int32),
        pltpu.VMEM((ACH, D), jnp.float32),
        pltpu.VMEM((ACH, D), jnp.float32),
        pltpu.VMEM((ACH, D), jnp.float32),
        pltpu.VMEM_SHARED((N_PAD, D), jnp.float32),
        pltpu.SemaphoreType.DMA,
        pltpu.SemaphoreType.DMA,
        pltpu.SemaphoreType.DMA,
        pltpu.SemaphoreType.DMA,
        pltpu.SemaphoreType.DMA,
        pltpu.SemaphoreType.DMA,
    ],
)
def _agg_kernel(h_hbm, src_hbm, dst_hbm, out_hbm,
                dst_v, src_0, src_1, src_2, rows_a, rows_b, rows_c, acc_sh,
                sem_a, sem_b, sem_c, sem_s0, sem_s1, sem_s2):
    c = lax.axis_index("c")
    s = lax.axis_index("s")
    wid = c * NS + s

    def zrow(i, carry):
        for k in range(D // 16):
            rows_a[i, pl.ds(k * 16, 16)] = jnp.zeros((16,), jnp.float32)
        return carry

    lax.fori_loop(0, ACH, zrow, 0)
    for t in range(RPT // ACH):
        pltpu.sync_copy(rows_a, acc_sh.at[pl.ds(s * RPT + t * ACH, ACH)])
    pltpu.sync_copy(dst_hbm.at[wid], dst_v)

    def fetch_src(j, buf, sem):
        pltpu.async_copy(src_hbm.at[wid, j], buf, sem)

    def wait_src(j, buf, sem):
        pltpu.make_async_copy(src_hbm.at[wid, j], buf, sem).wait()

    def gather(buf, rows, sem):
        pltpu.async_copy(h_hbm.at[buf], rows, sem)

    def wait_rows(buf, rows, sem):
        pltpu.make_async_copy(h_hbm.at[buf], rows, sem).wait()

    fetch_src(0, src_0, sem_s0)
    fetch_src(1, src_1, sem_s1)
    fetch_src(2, src_2, sem_s2)
    plsc.subcore_barrier()

    wait_src(0, src_0, sem_s0)
    gather(src_0, rows_a, sem_a)
    wait_src(1, src_1, sem_s1)
    gather(src_1, rows_b, sem_b)

    srcs = (src_0, src_1, src_2)
    rows = (rows_a, rows_b, rows_c)
    sems = (sem_a, sem_b, sem_c)
    ssems = (sem_s0, sem_s1, sem_s2)

    def body(t, carry):
        j0 = t * 3
        for k in range(3):
            j = j0 + k
            sl = k
            la = (k + 2) % 3

            @pl.when(j < ANCHUNK)
            def _():
                @pl.when(j + 2 < ANCHUNK)
                def _():
                    wait_src(j + 2, srcs[la], ssems[la])
                    gather(srcs[la], rows[la], sems[la])

                wait_rows(srcs[sl], rows[sl], sems[sl])
                pltpu.sync_copy(rows[sl], acc_sh.at[dst_v.at[j]], add=True)

                @pl.when(j + 3 < ANCHUNK)
                def _():
                    fetch_src(j + 3, srcs[sl], ssems[sl])
        return carry

    lax.fori_loop(0, pl.cdiv(ANCHUNK, 3), body, 0)
    plsc.subcore_barrier()
    pltpu.sync_copy(acc_sh.at[pl.ds(s * RPT, RPT)],
                    out_hbm.at[c, pl.ds(s * RPT, RPT)])


BR = 1024
GRID = N_PAD // BR


def _dinv_block(deg_ref, i):
    deg = (deg_ref[0:1, pl.ds(i * BR, BR)]
           + deg_ref[1:2, pl.ds(i * BR, BR)])
    dinv = lax.rsqrt(jnp.maximum(deg, 1.0))
    return jnp.transpose(dinv)


def _tc1_body(deg_ref, x_ref, w_ref, out_ref):
    dinv = _dinv_block(deg_ref, pl.program_id(0))
    out_ref[...] = jnp.dot(x_ref[...], w_ref[...],
                           preferred_element_type=jnp.float32) * dinv


def _tc2_body(deg_ref, p_ref, b_ref, w_ref, out_ref):
    dinv = _dinv_block(deg_ref, pl.program_id(0))
    agg = p_ref[0] + p_ref[1]
    x2 = jnp.maximum(agg * dinv + b_ref[...], 0.0)
    out_ref[...] = jnp.dot(x2, w_ref[...],
                           preferred_element_type=jnp.float32) * dinv


def _tc3_body(deg_ref, p_ref, b_ref, m1_ref, mb1_ref, m2_ref, mb2_ref,
              m3_ref, mb3_ref, out_ref, acc_ref):
    i = pl.program_id(0)
    dinv = _dinv_block(deg_ref, i)
    agg = p_ref[0] + p_ref[1]
    x3 = jnp.maximum(agg * dinv + b_ref[...], 0.0)
    row_ids = lax.broadcasted_iota(jnp.int32, (BR, 1), 0) + i * BR
    x3 = jnp.where(row_ids < N, x3, 0.0)
    part = jnp.sum(x3, axis=0, keepdims=True)

    @pl.when(i == 0)
    def _():
        acc_ref[...] = part

    @pl.when(i > 0)
    def _():
        acc_ref[...] = acc_ref[...] + part

    @pl.when(i == GRID - 1)
    def _():
        pooled = acc_ref[...] * (1.0 / N)
        z = jnp.maximum(jnp.dot(pooled, m1_ref[...],
                                preferred_element_type=jnp.float32)
                        + mb1_ref[...], 0.0)
        z = jnp.maximum(jnp.dot(z, m2_ref[...],
                                preferred_element_type=jnp.float32)
                        + mb2_ref[...], 0.0)
        out_ref[...] = jnp.dot(z, m3_ref[...],
                               preferred_element_type=jnp.float32) + mb3_ref[...]


def _deg_spec():
    return pl.BlockSpec((NC, N_PAD), lambda i: (0, 0))


def _p_spec():
    return pl.BlockSpec((NC, BR, D), lambda i: (0, i, 0))


def _full(shape):
    return pl.BlockSpec(shape, lambda i: tuple(0 for _ in shape))


_tc1 = pl.pallas_call(
    _tc1_body,
    grid=(GRID,),
    in_specs=[_deg_spec(), pl.BlockSpec((BR, D), lambda i: (i, 0)),
              _full((D, D))],
    out_specs=pl.BlockSpec((BR, D), lambda i: (i, 0)),
    out_shape=jax.ShapeDtypeStruct((N_PAD, D), jnp.float32),
)

_tc2 = pl.pallas_call(
    _tc2_body,
    grid=(GRID,),
    in_specs=[_deg_spec(), _p_spec(), _full((1, D)), _full((D, D))],
    out_specs=pl.BlockSpec((BR, D), lambda i: (i, 0)),
    out_shape=jax.ShapeDtypeStruct((N_PAD, D), jnp.float32),
)

_tc3 = pl.pallas_call(
    _tc3_body,
    grid=(GRID,),
    in_specs=[_deg_spec(), _p_spec(), _full((1, D)),
              _full((D, D)), _full((1, D)),
              _full((D, D)), _full((1, D)),
              _full((D, 1)), _full((1, 1))],
    out_specs=_full((1, 1)),
    out_shape=jax.ShapeDtypeStruct((1, 1), jnp.float32),
    scratch_shapes=[pltpu.VMEM((1, D), jnp.float32)],
)


def kernel(x, edge_index, W1, b1, W2, b2, M1, mb1, M2, mb2, M3, mb3):
    dst = edge_index[1].reshape(NW, NCHUNK, CH)
    src_p = edge_index[0].reshape(NW, ANCHUNK, ACH)
    dst_p = edge_index[1].reshape(NW, ANCHUNK, ACH)
    z1 = jnp.zeros((N_PAD,), jnp.float32)

    deg2 = _deg_kernel(dst, z1, jnp.ones((CH,), jnp.float32))

    x_p = jnp.pad(x, ((0, N_PAD - N), (0, 0)))
    h1 = _tc1(deg2, x_p, W1)
    p1 = _agg_kernel(h1, src_p, dst_p)
    h2 = _tc2(deg2, p1, b1.reshape(1, D), W2)
    p2 = _agg_kernel(h2, src_p, dst_p)
    out = _tc3(deg2, p2, b2.reshape(1, D),
               M1, mb1.reshape(1, D), M2, mb2.reshape(1, D),
               M3.reshape(D, 1), mb3.reshape(1, 1))
    return out


# speedup vs baseline: 1.0863x; 1.0863x over previous
import functools

import jax
import jax.numpy as jnp
from jax import lax
from jax.experimental import pallas as pl
from jax.experimental.pallas import tpu as pltpu
from jax.experimental.pallas import tpu_sc as plsc

N = 10000
E = 320000
D = 128

NC = 2
NS = 16
NW = NC * NS
EPT = E // NW
CH = 80
NCHUNK = EPT // CH

ACH = 80
ANCHUNK = 125
N_PAD = 10240
RPT = N_PAD // NS

_MESH = plsc.VectorSubcoreMesh(core_axis_name="c", subcore_axis_name="s")


@functools.partial(
    pl.kernel,
    out_type=jax.ShapeDtypeStruct((NC, N_PAD), jnp.float32),
    mesh=_MESH,
    scratch_types=[
        pltpu.VMEM((NCHUNK, CH), jnp.int32),
        pltpu.VMEM((CH,), jnp.float32),
        pltpu.VMEM_SHARED((N_PAD,), jnp.float32),
    ],
)
def _deg_kernel(dst_hbm, z1_hbm, ones_hbm, out_hbm, idx_v, ones_v, deg_sh):
    c = lax.axis_index("c")
    s = lax.axis_index("s")
    wid = c * NS + s
    pltpu.sync_copy(z1_hbm.at[pl.ds(s * RPT, RPT)],
                    deg_sh.at[pl.ds(s * RPT, RPT)])
    pltpu.sync_copy(dst_hbm.at[wid], idx_v)
    pltpu.sync_copy(ones_hbm, ones_v)
    plsc.subcore_barrier()

    def body(j, carry):
        pltpu.sync_copy(ones_v, deg_sh.at[idx_v.at[j]], add=True)
        return carry

    lax.fori_loop(0, NCHUNK, body, 0)
    plsc.subcore_barrier()
    pltpu.sync_copy(deg_sh.at[pl.ds(s * RPT, RPT)],
                    out_hbm.at[c, pl.ds(s * RPT, RPT)])


@functools.partial(
    pl.kernel,
    out_type=jax.ShapeDtypeStruct((NC, N_PAD, D), jnp.float32),
    mesh=_MESH,
    scratch_types=[
        pltpu.VMEM((ANCHUNK, ACH), jnp.int32),
        pltpu.VMEM((ACH,), jnp.int32),
        pltpu.VMEM((ACH,), jnp.int32),
        pltpu.VMEM((ACH, D), jnp.float32),
        pltpu.VMEM((ACH, D), jnp.float32),
        pltpu.VMEM_SHARED((N_PAD, D), jnp.float32),
        pltpu.SemaphoreType.DMA,
        pltpu.SemaphoreType.DMA,
        pltpu.SemaphoreType.DMA,
        pltpu.SemaphoreType.DMA,
        pltpu.SemaphoreType.DMA,
        pltpu.SemaphoreType.DMA,
    ],
)
def _agg_kernel(h_hbm, src_hbm, dst_hbm, out_hbm,
                dst_v, src_0, src_1, rows_a, rows_b, acc_sh,
                sem_a, sem_b, sem_s0, sem_s1, sem_wa, sem_wb):
    c = lax.axis_index("c")
    s = lax.axis_index("s")
    wid = c * NS + s

    def zrow(i, carry):
        for k in range(D // 16):
            rows_a[i, pl.ds(k * 16, 16)] = jnp.zeros((16,), jnp.float32)
        return carry

    lax.fori_loop(0, ACH, zrow, 0)
    for t in range(RPT // ACH):
        pltpu.sync_copy(rows_a, acc_sh.at[pl.ds(s * RPT + t * ACH, ACH)])
    pltpu.sync_copy(dst_hbm.at[wid], dst_v)

    def fetch_src(j, buf, sem):
        return pltpu.async_copy(src_hbm.at[wid, j], buf, sem)

    fetch_src(0, src_0, sem_s0)
    fetch_src(1, src_1, sem_s1)
    plsc.subcore_barrier()

    pltpu.make_async_copy(src_hbm.at[wid, 0], src_0, sem_s0).wait()
    pltpu.async_copy(h_hbm.at[src_0], rows_a, sem_a)

    def wait_scat(rows, sem):
        pltpu.make_async_copy(rows, acc_sh.at[dst_v.at[0]], sem).wait()

    def body(jj, carry):
        j0 = jj * 2
        j1 = j0 + 1

        @pl.when(jj > 0)
        def _():
            wait_scat(rows_b, sem_wb)

        pltpu.make_async_copy(src_hbm.at[wid, j1], src_1, sem_s1).wait()
        pltpu.async_copy(h_hbm.at[src_1], rows_b, sem_b)

        pltpu.make_async_copy(h_hbm.at[src_0], rows_a, sem_a).wait()
        pltpu.async_copy(rows_a, acc_sh.at[dst_v.at[j0]], sem_wa, add=True)

        @pl.when(j0 + 2 < ANCHUNK)
        def _():
            fetch_src(j0 + 2, src_0, sem_s0)

        wait_scat(rows_a, sem_wa)

        @pl.when(j0 + 2 < ANCHUNK)
        def _():
            pltpu.make_async_copy(src_hbm.at[wid, j0 + 2], src_0,
                                  sem_s0).wait()
            pltpu.async_copy(h_hbm.at[src_0], rows_a, sem_a)

        @pl.when(j1 + 2 < ANCHUNK)
        def _():
            fetch_src(j1 + 2, src_1, sem_s1)

        pltpu.make_async_copy(h_hbm.at[src_1], rows_b, sem_b).wait()
        pltpu.async_copy(rows_b, acc_sh.at[dst_v.at[j1]], sem_wb, add=True)
        return carry

    lax.fori_loop(0, ANCHUNK // 2, body, 0)
    wait_scat(rows_b, sem_wb)
    pltpu.make_async_copy(h_hbm.at[src_0], rows_a, sem_a).wait()
    pltpu.sync_copy(rows_a, acc_sh.at[dst_v.at[ANCHUNK - 1]], add=True)
    plsc.subcore_barrier()
    pltpu.sync_copy(acc_sh.at[pl.ds(s * RPT, RPT)],
                    out_hbm.at[c, pl.ds(s * RPT, RPT)])


BR = 1024
GRID = N_PAD // BR


def _dinv_block(deg_ref, i):
    deg = (deg_ref[0:1, pl.ds(i * BR, BR)]
           + deg_ref[1:2, pl.ds(i * BR, BR)])
    dinv = lax.rsqrt(jnp.maximum(deg, 1.0))
    return jnp.transpose(dinv)


def _tc1_body(deg_ref, x_ref, w_ref, out_ref):
    dinv = _dinv_block(deg_ref, pl.program_id(0))
    out_ref[...] = jnp.dot(x_ref[...], w_ref[...],
                           preferred_element_type=jnp.float32) * dinv


def _tc2_body(deg_ref, p_ref, b_ref, w_ref, out_ref):
    dinv = _dinv_block(deg_ref, pl.program_id(0))
    agg = p_ref[0] + p_ref[1]
    x2 = jnp.maximum(agg * dinv + b_ref[...], 0.0)
    out_ref[...] = jnp.dot(x2, w_ref[...],
                           preferred_element_type=jnp.float32) * dinv


def _tc3_body(deg_ref, p_ref, b_ref, m1_ref, mb1_ref, m2_ref, mb2_ref,
              m3_ref, mb3_ref, out_ref, acc_ref):
    i = pl.program_id(0)
    dinv = _dinv_block(deg_ref, i)
    agg = p_ref[0] + p_ref[1]
    x3 = jnp.maximum(agg * dinv + b_ref[...], 0.0)
    row_ids = lax.broadcasted_iota(jnp.int32, (BR, 1), 0) + i * BR
    x3 = jnp.where(row_ids < N, x3, 0.0)
    part = jnp.sum(x3, axis=0, keepdims=True)

    @pl.when(i == 0)
    def _():
        acc_ref[...] = part

    @pl.when(i > 0)
    def _():
        acc_ref[...] = acc_ref[...] + part

    @pl.when(i == GRID - 1)
    def _():
        pooled = acc_ref[...] * (1.0 / N)
        z = jnp.maximum(jnp.dot(pooled, m1_ref[...],
                                preferred_element_type=jnp.float32)
                        + mb1_ref[...], 0.0)
        z = jnp.maximum(jnp.dot(z, m2_ref[...],
                                preferred_element_type=jnp.float32)
                        + mb2_ref[...], 0.0)
        out_ref[...] = jnp.dot(z, m3_ref[...],
                               preferred_element_type=jnp.float32) + mb3_ref[...]


def _deg_spec():
    return pl.BlockSpec((NC, N_PAD), lambda i: (0, 0))


def _p_spec():
    return pl.BlockSpec((NC, BR, D), lambda i: (0, i, 0))


def _full(shape):
    return pl.BlockSpec(shape, lambda i: tuple(0 for _ in shape))


_tc1 = pl.pallas_call(
    _tc1_body,
    grid=(GRID,),
    in_specs=[_deg_spec(), pl.BlockSpec((BR, D), lambda i: (i, 0)),
              _full((D, D))],
    out_specs=pl.BlockSpec((BR, D), lambda i: (i, 0)),
    out_shape=jax.ShapeDtypeStruct((N_PAD, D), jnp.float32),
)

_tc2 = pl.pallas_call(
    _tc2_body,
    grid=(GRID,),
    in_specs=[_deg_spec(), _p_spec(), _full((1, D)), _full((D, D))],
    out_specs=pl.BlockSpec((BR, D), lambda i: (i, 0)),
    out_shape=jax.ShapeDtypeStruct((N_PAD, D), jnp.float32),
)

_tc3 = pl.pallas_call(
    _tc3_body,
    grid=(GRID,),
    in_specs=[_deg_spec(), _p_spec(), _full((1, D)),
              _full((D, D)), _full((1, D)),
              _full((D, D)), _full((1, D)),
              _full((D, 1)), _full((1, 1))],
    out_specs=_full((1, 1)),
    out_shape=jax.ShapeDtypeStruct((1, 1), jnp.float32),
    scratch_shapes=[pltpu.VMEM((1, D), jnp.float32)],
)


def kernel(x, edge_index, W1, b1, W2, b2, M1, mb1, M2, mb2, M3, mb3):
    dst = edge_index[1].reshape(NW, NCHUNK, CH)
    src_p = edge_index[0].reshape(NW, ANCHUNK, ACH)
    dst_p = edge_index[1].reshape(NW, ANCHUNK, ACH)
    z1 = jnp.zeros((N_PAD,), jnp.float32)

    deg2 = _deg_kernel(dst, z1, jnp.ones((CH,), jnp.float32))

    x_p = jnp.pad(x, ((0, N_PAD - N), (0, 0)))
    h1 = _tc1(deg2, x_p, W1)
    p1 = _agg_kernel(h1, src_p, dst_p)
    h2 = _tc2(deg2, p1, b1.reshape(1, D), W2)
    p2 = _agg_kernel(h2, src_p, dst_p)
    out = _tc3(deg2, p2, b2.reshape(1, D),
               M1, mb1.reshape(1, D), M2, mb2.reshape(1, D),
               M3.reshape(D, 1), mb3.reshape(1, 1))
    return out
